# Initial kernel scaffold; baseline (speedup 1.0000x reference)
import jax
import jax.numpy as jnp
from jax.experimental import pallas as pl


def kernel(x_real, x_imag, xp, yp):
    raise NotImplementedError("write your pallas kernel here")


# speedup vs baseline: 577.7719x; 577.7719x over previous
import functools

import jax
import jax.numpy as jnp
import numpy as np
from jax import lax
from jax.experimental import pallas as pl
from jax.experimental.pallas import tpu as pltpu
from jax.experimental.pallas import tpu_sc as plsc

NUM_CHANNELS = 128
NUM_BREAKPOINTS = 101
NUM_SEG = NUM_BREAKPOINTS - 1
TAB = NUM_CHANNELS * NUM_SEG
B, C, L = 8, NUM_CHANNELS, 16384
ROWS = B * C
NUM_WORKERS = 32
ROWS_PER_W = ROWS // NUM_WORKERS
VECS = L // 16

INV_H = np.float32(NUM_SEG / 6.0)
OFF = np.float32(NUM_SEG / 2.0)


def _pwl_sc(xr2d, xi2d, ab):
    mesh = plsc.VectorSubcoreMesh(core_axis_name="c", subcore_axis_name="s")

    @functools.partial(
        pl.kernel,
        mesh=mesh,
        out_type=jax.ShapeDtypeStruct((ROWS, 2 * L), jnp.float32),
        scratch_types=[
            pltpu.VMEM((2 * TAB,), jnp.float32),
            pltpu.VMEM((L,), jnp.float32),
            pltpu.VMEM((L,), jnp.float32),
            pltpu.VMEM((2 * L,), jnp.float32),
        ],
        compiler_params=pltpu.CompilerParams(needs_layout_passes=False),
    )
    def k(xr_hbm, xi_hbm, ab_hbm, out_hbm, ab_v, xr_v, xi_v, o_v):
        wid = lax.axis_index("s") * 2 + lax.axis_index("c")
        pltpu.sync_copy(ab_hbm, ab_v)
        base_row = wid * ROWS_PER_W
        iota = lax.iota(jnp.int32, 16)

        def row_body(j, _):
            r = base_row + j
            tab_base = lax.rem(r, NUM_CHANNELS) * NUM_SEG
            pltpu.sync_copy(xr_hbm.at[r], xr_v)
            pltpu.sync_copy(xi_hbm.at[r], xi_v)

            def vec_body(i, _):
                xr = xr_v[pl.ds(i * 16, 16)]
                xi = xi_v[pl.ds(i * 16, 16)]
                tr = jnp.maximum(jnp.minimum(xr * INV_H + OFF, 99.0), 0.0)
                ti = jnp.maximum(jnp.minimum(xi * INV_H + OFF, 99.0), 0.0)
                sr = tr.astype(jnp.int32) + tab_base
                si = ti.astype(jnp.int32) + tab_base
                ar = plsc.load_gather(ab_v, [sr])
                br = plsc.load_gather(ab_v, [sr + TAB])
                ai = plsc.load_gather(ab_v, [si])
                bi = plsc.load_gather(ab_v, [si + TAB])
                yr = ar * xr + br
                yi = ai * xi + bi
                oe = i * 32 + 2 * iota
                plsc.store_scatter(o_v, [oe], yr)
                plsc.store_scatter(o_v, [oe + 1], yi)
                return 0

            lax.fori_loop(0, VECS, vec_body, 0)
            pltpu.sync_copy(o_v, out_hbm.at[r])
            return 0

        lax.fori_loop(0, ROWS_PER_W, row_body, 0)

    return k(xr2d, xi2d, ab)


@jax.jit
def kernel(x_real, x_imag, xp, yp):
    a = (yp[:, 1:] - yp[:, :-1]) / (xp[:, 1:] - xp[:, :-1])
    b = yp[:, :-1] - a * xp[:, :-1]
    ab = jnp.concatenate([a.reshape(-1), b.reshape(-1)])
    out = _pwl_sc(x_real.reshape(ROWS, L), x_imag.reshape(ROWS, L), ab)
    return out.reshape(B, C, 2 * L).view(jnp.complex64)


# speedup vs baseline: 604.3723x; 1.0460x over previous
import functools

import jax
import jax.numpy as jnp
import numpy as np
from jax import lax
from jax.experimental import pallas as pl
from jax.experimental.pallas import tpu as pltpu
from jax.experimental.pallas import tpu_sc as plsc

NUM_CHANNELS = 128
NUM_BREAKPOINTS = 101
NUM_SEG = NUM_BREAKPOINTS - 1
TAB = NUM_CHANNELS * NUM_SEG
B, C, L = 8, NUM_CHANNELS, 16384
ROWS = B * C
NUM_WORKERS = 32
ROWS_PER_W = ROWS // NUM_WORKERS
VECS = L // 16

INV_H = np.float32(NUM_SEG / 6.0)
OFF = np.float32(NUM_SEG / 2.0)


def _pwl_sc(xr2d, xi2d, ab):
    mesh = plsc.VectorSubcoreMesh(core_axis_name="c", subcore_axis_name="s")

    @functools.partial(
        pl.kernel,
        mesh=mesh,
        out_type=jax.ShapeDtypeStruct((ROWS, 2 * L), jnp.float32),
        scratch_types=[
            pltpu.VMEM((2 * TAB,), jnp.float32),
            pltpu.VMEM((L,), jnp.float32),
            pltpu.VMEM((L,), jnp.float32),
            pltpu.VMEM((2 * L,), jnp.float32),
        ],
        compiler_params=pltpu.CompilerParams(needs_layout_passes=False),
    )
    def k(xr_hbm, xi_hbm, ab_hbm, out_hbm, ab_v, xr_v, xi_v, o_v):
        wid = lax.axis_index("s") * 2 + lax.axis_index("c")
        pltpu.sync_copy(ab_hbm, ab_v)
        base_row = wid * ROWS_PER_W
        iota = lax.iota(jnp.int32, 16)

        def row_body(j, _):
            r = base_row + j
            tab_base = lax.rem(r, NUM_CHANNELS) * NUM_SEG
            pltpu.sync_copy(xr_hbm.at[r], xr_v)
            pltpu.sync_copy(xi_hbm.at[r], xi_v)

            @plsc.parallel_loop(0, VECS, unroll=8)
            def vec_body(i):
                xr = xr_v[pl.ds(i * 16, 16)]
                xi = xi_v[pl.ds(i * 16, 16)]
                tr = jnp.maximum(jnp.minimum(xr * INV_H + OFF, 99.0), 0.0)
                ti = jnp.maximum(jnp.minimum(xi * INV_H + OFF, 99.0), 0.0)
                sr = tr.astype(jnp.int32) + tab_base
                si = ti.astype(jnp.int32) + tab_base
                ar = plsc.load_gather(ab_v, [sr])
                br = plsc.load_gather(ab_v, [sr + TAB])
                ai = plsc.load_gather(ab_v, [si])
                bi = plsc.load_gather(ab_v, [si + TAB])
                yr = ar * xr + br
                yi = ai * xi + bi
                oe = i * 32 + 2 * iota
                plsc.store_scatter(o_v, [oe], yr)
                plsc.store_scatter(o_v, [oe + 1], yi)
            pltpu.sync_copy(o_v, out_hbm.at[r])
            return 0

        lax.fori_loop(0, ROWS_PER_W, row_body, 0)

    return k(xr2d, xi2d, ab)


@jax.jit
def kernel(x_real, x_imag, xp, yp):
    a = (yp[:, 1:] - yp[:, :-1]) / (xp[:, 1:] - xp[:, :-1])
    b = yp[:, :-1] - a * xp[:, :-1]
    ab = jnp.concatenate([a.reshape(-1), b.reshape(-1)])
    out = _pwl_sc(x_real.reshape(ROWS, L), x_imag.reshape(ROWS, L), ab)
    return out.reshape(B, C, 2 * L).view(jnp.complex64)


# speedup vs baseline: 2826.0877x; 4.6761x over previous
import functools

import jax
import jax.numpy as jnp
import numpy as np
from jax import lax
from jax.experimental import pallas as pl
from jax.experimental.pallas import tpu as pltpu
from jax.experimental.pallas import tpu_sc as plsc

NUM_CHANNELS = 128
NUM_BREAKPOINTS = 101
NUM_SEG = NUM_BREAKPOINTS - 1
TAB = NUM_CHANNELS * NUM_SEG
B, C, L = 8, NUM_CHANNELS, 16384
ROWS = B * C
NUM_WORKERS = 32
ROWS_PER_W = ROWS // NUM_WORKERS
VECS = L // 16

INV_H = np.float32(NUM_SEG / 6.0)
OFF = np.float32(NUM_SEG / 2.0)


def _pwl_sc(xr2d, xi2d, ab):
    mesh = plsc.VectorSubcoreMesh(core_axis_name="c", subcore_axis_name="s")

    @functools.partial(
        pl.kernel,
        mesh=mesh,
        out_type=(
            jax.ShapeDtypeStruct((ROWS, L), jnp.float32),
            jax.ShapeDtypeStruct((ROWS, L), jnp.float32),
        ),
        scratch_types=[
            pltpu.VMEM((2 * TAB,), jnp.float32),
            pltpu.VMEM((L,), jnp.float32),
            pltpu.VMEM((L,), jnp.float32),
            pltpu.VMEM((L,), jnp.float32),
            pltpu.VMEM((L,), jnp.float32),
        ],
        compiler_params=pltpu.CompilerParams(needs_layout_passes=False),
    )
    def k(xr_hbm, xi_hbm, ab_hbm, yr_hbm, yi_hbm, ab_v, xr_v, xi_v, or_v, oi_v):
        wid = lax.axis_index("s") * 2 + lax.axis_index("c")
        pltpu.sync_copy(ab_hbm, ab_v)
        base_row = wid * ROWS_PER_W

        def row_body(j, _):
            r = base_row + j
            tab_base = lax.rem(r, NUM_CHANNELS) * NUM_SEG
            pltpu.sync_copy(xr_hbm.at[r], xr_v)
            pltpu.sync_copy(xi_hbm.at[r], xi_v)

            @plsc.parallel_loop(0, VECS, unroll=8)
            def vec_body(i):
                xr = xr_v[pl.ds(i * 16, 16)]
                xi = xi_v[pl.ds(i * 16, 16)]
                tr = jnp.maximum(jnp.minimum(xr * INV_H + OFF, 99.0), 0.0)
                ti = jnp.maximum(jnp.minimum(xi * INV_H + OFF, 99.0), 0.0)
                sr = tr.astype(jnp.int32) + tab_base
                si = ti.astype(jnp.int32) + tab_base
                ar = plsc.load_gather(ab_v, [sr])
                br = plsc.load_gather(ab_v, [sr + TAB])
                ai = plsc.load_gather(ab_v, [si])
                bi = plsc.load_gather(ab_v, [si + TAB])
                or_v[pl.ds(i * 16, 16)] = ar * xr + br
                oi_v[pl.ds(i * 16, 16)] = ai * xi + bi
            pltpu.sync_copy(or_v, yr_hbm.at[r])
            pltpu.sync_copy(oi_v, yi_hbm.at[r])
            return 0

        lax.fori_loop(0, ROWS_PER_W, row_body, 0)

    return k(xr2d, xi2d, ab)


@jax.jit
def kernel(x_real, x_imag, xp, yp):
    a = (yp[:, 1:] - yp[:, :-1]) / (xp[:, 1:] - xp[:, :-1])
    b = yp[:, :-1] - a * xp[:, :-1]
    ab = jnp.concatenate([a.reshape(-1), b.reshape(-1)])
    yr, yi = _pwl_sc(x_real.reshape(ROWS, L), x_imag.reshape(ROWS, L), ab)
    return jax.lax.complex(yr.reshape(B, C, L), yi.reshape(B, C, L))
